# Initial kernel scaffold; baseline (speedup 1.0000x reference)
import jax
import jax.numpy as jnp
from jax.experimental import pallas as pl


def kernel(x_nchw, w1, b1, w2, b2, wfc1, bfc1, wfc2, bfc2):
    raise NotImplementedError("write your pallas kernel here")


# speedup vs baseline: 211.3498x; 211.3498x over previous
import functools

import jax
import jax.numpy as jnp
from jax.experimental import pallas as pl
from jax.experimental.pallas import tpu as pltpu

_B = 64
_H, _W, _CIN = 28, 28, 3
_H1, _W1C = 26, 896
_H2, _W2C = 24, 1536
_KP = 1472


def _body(x0_ref, w1_ref, b1_ref, w2_ref, b2_ref, wfc1_ref, bfc1_ref,
          wfc2_ref, bfc2_ref, o_ref, acc1_ref, x1_ref, x2_ref):
    B = x0_ref.shape[1]
    for dh in range(3):
        lhs = x0_ref[dh:dh + _H1].reshape(_H1 * B, _W * _CIN)
        d = jnp.dot(lhs, w1_ref[dh], preferred_element_type=jnp.float32)
        if dh == 0:
            acc1_ref[...] = d
        else:
            acc1_ref[...] += d
    x1_ref[...] = jnp.maximum(acc1_ref[...] + b1_ref[...], 0.0).astype(
        jnp.bfloat16).reshape(_H1, B, _W1C)

    for g in range(6):
        acc = None
        for dh in range(3):
            lhs = x1_ref[dh:dh + _H2, :, 128 * g:128 * g + 256]
            d = jnp.dot(lhs.reshape(_H2 * B, 256), w2_ref[dh],
                        preferred_element_type=jnp.float32)
            acc = d if acc is None else acc + d
        x2_ref[:, :, 256 * g:256 * (g + 1)] = jnp.maximum(
            acc + b2_ref[:, 256 * g:256 * (g + 1)], 0.0).astype(
            jnp.bfloat16).reshape(_H2, B, 256)

    accf = None
    for hp in range(12):
        r0 = jnp.maximum(x2_ref[2 * hp, :, 0:_KP], x2_ref[2 * hp, :, 64:])
        r1 = jnp.maximum(x2_ref[2 * hp + 1, :, 0:_KP],
                         x2_ref[2 * hp + 1, :, 64:])
        ye = jnp.maximum(r0, r1)
        d = jnp.dot(ye, wfc1_ref[hp], preferred_element_type=jnp.float32)
        accf = d if accf is None else accf + d
    h = jnp.maximum(accf + bfc1_ref[...], 0.0)
    o_ref[...] = (jnp.dot(h, wfc2_ref[...],
                          preferred_element_type=jnp.float32)
                  + bfc2_ref[...])


def kernel(x_nchw, w1, b1, w2, b2, wfc1, bfc1, wfc2, bfc2):
    N = x_nchw.shape[0]
    f32, bf16 = jnp.float32, jnp.bfloat16

    x0 = jnp.transpose(x_nchw, (2, 0, 3, 1)).reshape(_H, N, _W * _CIN)
    x0 = x0.astype(bf16)

    w1r = w1.reshape(3, 3, _CIN, 32)
    w1s = []
    for dh in range(3):
        t = sum(jnp.eye(_H, _H1, k=-dw, dtype=f32)[:, None, :, None]
                * w1r[dh, dw][None, :, None, :] for dw in range(3))
        w1s.append(jnp.pad(t.reshape(_W * _CIN, _H1 * 32), ((0, 0), (0, 64))))
    w1s = jnp.stack(w1s).astype(bf16)
    b1big = jnp.pad(jnp.tile(b1, _H1), (0, 64)).reshape(1, _W1C)

    w2r = w2.reshape(3, 3, 32, 64)
    w2s = []
    for dh in range(3):
        t = sum(jnp.eye(8, 4, k=-dw, dtype=f32)[:, None, :, None]
                * w2r[dh, dw][None, :, None, :] for dw in range(3))
        w2s.append(t.reshape(256, 256))
    w2s = jnp.stack(w2s).astype(bf16)
    b2big = jnp.tile(b2, _H2).reshape(1, _W2C)

    wr = wfc1.reshape(12, 12, 64, 128)
    z = jnp.zeros((12, 23, 64, 128), f32).at[:, 0::2].set(wr)
    wfc1e = z.reshape(12, _KP, 128).astype(bf16)

    grid = (N // _B,)
    out = pl.pallas_call(
        _body,
        out_shape=jax.ShapeDtypeStruct((N, 9), f32),
        grid_spec=pltpu.PrefetchScalarGridSpec(
            num_scalar_prefetch=0,
            grid=grid,
            in_specs=[
                pl.BlockSpec((_H, _B, _W * _CIN), lambda i: (0, i, 0)),
                pl.BlockSpec((3, _W * _CIN, _W1C), lambda i: (0, 0, 0)),
                pl.BlockSpec((1, _W1C), lambda i: (0, 0)),
                pl.BlockSpec((3, 256, 256), lambda i: (0, 0, 0)),
                pl.BlockSpec((1, _W2C), lambda i: (0, 0)),
                pl.BlockSpec((12, _KP, 128), lambda i: (0, 0, 0)),
                pl.BlockSpec((1, 128), lambda i: (0, 0)),
                pl.BlockSpec((128, 9), lambda i: (0, 0)),
                pl.BlockSpec((1, 9), lambda i: (0, 0)),
            ],
            out_specs=pl.BlockSpec((_B, 9), lambda i: (i, 0)),
            scratch_shapes=[
                pltpu.VMEM((_H1 * _B, _W1C), f32),
                pltpu.VMEM((_H1, _B, _W1C), bf16),
                pltpu.VMEM((_H2, _B, _W2C), bf16),
            ],
        ),
        compiler_params=pltpu.CompilerParams(
            dimension_semantics=("parallel",)),
    )(x0, w1s, b1big, w2s, b2big, wfc1e,
      bfc1.reshape(1, 128), wfc2, bfc2.reshape(1, 9))
    return out
